# Initial kernel scaffold; baseline (speedup 1.0000x reference)
import jax
import jax.numpy as jnp
from jax.experimental import pallas as pl


def kernel(x, edge_index, adj_values, gc1_W, gc1_b, gc2_W, gc2_b, cls_W, cls_b):
    raise NotImplementedError("write your pallas kernel here")


# speedup vs baseline: 2.5930x; 2.5930x over previous
import functools

import jax
import jax.numpy as jnp
from jax import lax
from jax.experimental import pallas as pl
from jax.experimental.pallas import tpu as pltpu
from jax.experimental.pallas import tpu_sc as plsc

N_NODES = 10000
N_EDGES = 320000
D_IN = 128
D_HID = 128
N_CLASSES = 40

NC = 2
NS = 16
LANES = 16
NW = NC * NS
FPW = D_HID // NW
CH = 3200
NCH = N_EDGES // CH
GR = CH // LANES

_SC_MESH = plsc.VectorSubcoreMesh(
    core_axis_name="c", subcore_axis_name="s", num_cores=NC, num_subcores=NS)


@functools.partial(
    pl.kernel,
    out_type=jax.ShapeDtypeStruct((D_HID, N_NODES), jnp.float32),
    mesh=_SC_MESH,
    compiler_params=pltpu.CompilerParams(needs_layout_passes=False),
    scratch_types=[
        pltpu.VMEM((FPW, N_NODES), jnp.float32),
        pltpu.VMEM((FPW, N_NODES), jnp.float32),
        pltpu.VMEM((2, CH), jnp.int32),
        pltpu.VMEM((2, CH), jnp.int32),
        pltpu.VMEM((2, CH), jnp.float32),
        pltpu.SemaphoreType.DMA,
        pltpu.SemaphoreType.DMA,
    ],
)
def _spmm_sc(tab_hbm, row_hbm, col_hbm, val_hbm, out_hbm,
             tab_v, acc_v, rowb, colb, valb, sem0, sem1):
    c = lax.axis_index("c")
    s = lax.axis_index("s")
    wid = s * NC + c
    f0 = wid * FPW
    sems = (sem0, sem1)

    def _copies(ci, b):
        off = ci * CH
        return (
            pltpu.make_async_copy(row_hbm.at[pl.ds(off, CH)], rowb.at[b], sems[b]),
            pltpu.make_async_copy(col_hbm.at[pl.ds(off, CH)], colb.at[b], sems[b]),
            pltpu.make_async_copy(val_hbm.at[pl.ds(off, CH)], valb.at[b], sems[b]),
        )

    for cp in _copies(0, 0):
        cp.start()

    pltpu.sync_copy(tab_hbm.at[pl.ds(f0, FPW)], tab_v)

    zero = jnp.zeros((LANES,), jnp.float32)

    def _zbody(i, carry):
        for d in range(FPW):
            acc_v[d, pl.ds(i * LANES, LANES)] = zero
        return carry

    lax.fori_loop(0, N_NODES // LANES, _zbody, 0)

    def _process(b):
        def _gbody(g, carry):
            sl = pl.ds(g * LANES, LANES)
            row = rowb[b, sl]
            col = colb[b, sl]
            val = valb[b, sl]
            for d in range(FPW):
                dvec = jnp.full((LANES,), d, jnp.int32)
                gathered = plsc.load_gather(tab_v, [dvec, col])
                plsc.addupdate_scatter(acc_v, [dvec, row], gathered * val)
            return carry

        lax.fori_loop(0, GR, _gbody, 0)

    def _pair(i, carry):
        for b in range(2):
            ci = i * 2 + b

            @pl.when(ci + 1 < NCH)
            def _():
                for cp in _copies(ci + 1, 1 - b):
                    cp.start()

            for cp in _copies(ci, b):
                cp.wait()
            _process(b)
        return carry

    lax.fori_loop(0, NCH // 2, _pair, 0)

    pltpu.sync_copy(acc_v, out_hbm.at[pl.ds(f0, FPW)])


def _mm1_body(x_ref, w_ref, o_ref):
    o_ref[...] = lax.dot_general(
        w_ref[...], x_ref[...], (((0,), (1,)), ((), ())),
        preferred_element_type=jnp.float32)


def _dense1(x, W):
    return pl.pallas_call(
        _mm1_body,
        out_shape=jax.ShapeDtypeStruct((D_HID, N_NODES), jnp.float32),
    )(x, W)


def _mm2_body(p_ref, b_ref, w_ref, o_ref):
    h = jnp.maximum(p_ref[...] + b_ref[...], 0.0)
    o_ref[...] = lax.dot_general(
        w_ref[...], h, (((0,), (0,)), ((), ())),
        preferred_element_type=jnp.float32)


def _dense2(p, b, W):
    return pl.pallas_call(
        _mm2_body,
        out_shape=jax.ShapeDtypeStruct((D_HID, N_NODES), jnp.float32),
    )(p, b, W)


def _mm3_body(p_ref, b_ref, w_ref, cb_ref, o_ref):
    h2 = p_ref[...] + b_ref[...]
    logits = lax.dot_general(
        h2, w_ref[...], (((0,), (0,)), ((), ())),
        preferred_element_type=jnp.float32) + cb_ref[...]
    m = jnp.max(logits, axis=-1, keepdims=True)
    lse = jnp.log(jnp.sum(jnp.exp(logits - m), axis=-1, keepdims=True)) + m
    o_ref[...] = logits - lse


def _dense3(p, b, W, cb):
    return pl.pallas_call(
        _mm3_body,
        out_shape=jax.ShapeDtypeStruct((N_NODES, N_CLASSES), jnp.float32),
    )(p, b, W, cb)


def kernel(x, edge_index, adj_values, gc1_W, gc1_b, gc2_W, gc2_b, cls_W, cls_b):
    row = edge_index[0].astype(jnp.int32)
    col = edge_index[1].astype(jnp.int32)
    val = adj_values.astype(jnp.float32)
    b1 = gc1_b.reshape(D_HID, 1)
    b2 = gc2_b.reshape(D_HID, 1)
    cb = cls_b.reshape(1, N_CLASSES)

    s1T = _dense1(x, gc1_W)
    p1 = _spmm_sc(s1T, row, col, val)
    s2T = _dense2(p1, b1, gc2_W)
    p2 = _spmm_sc(s2T, row, col, val)
    return _dense3(p2, b2, cls_W, cb)


# speedup vs baseline: 2.7030x; 1.0424x over previous
import functools

import jax
import jax.numpy as jnp
from jax import lax
from jax.experimental import pallas as pl
from jax.experimental.pallas import tpu as pltpu
from jax.experimental.pallas import tpu_sc as plsc

N_NODES = 10000
N_EDGES = 320000
D_IN = 128
D_HID = 128
N_CLASSES = 40

NC = 2
NS = 16
LANES = 16
NW = NC * NS
FPW = D_HID // NW
CH = 3200
NCH = N_EDGES // CH
GR = CH // LANES

_SC_MESH = plsc.VectorSubcoreMesh(
    core_axis_name="c", subcore_axis_name="s", num_cores=NC, num_subcores=NS)


UNROLL = 4


@functools.partial(
    pl.kernel,
    out_type=jax.ShapeDtypeStruct((D_HID * N_NODES,), jnp.float32),
    mesh=_SC_MESH,
    compiler_params=pltpu.CompilerParams(needs_layout_passes=False),
    scratch_types=[
        pltpu.VMEM((FPW * N_NODES,), jnp.float32),
        pltpu.VMEM((FPW * N_NODES,), jnp.float32),
        pltpu.VMEM((2, CH), jnp.int32),
        pltpu.VMEM((2, CH), jnp.int32),
        pltpu.VMEM((2, CH), jnp.float32),
        pltpu.SemaphoreType.DMA,
        pltpu.SemaphoreType.DMA,
    ],
)
def _spmm_sc(tab_hbm, row_hbm, col_hbm, val_hbm, out_hbm,
             tab_v, acc_v, rowb, colb, valb, sem0, sem1):
    c = lax.axis_index("c")
    s = lax.axis_index("s")
    wid = s * NC + c
    e0 = wid * (FPW * N_NODES)
    sems = (sem0, sem1)

    def _copies(ci, b):
        off = ci * CH
        return (
            pltpu.make_async_copy(row_hbm.at[pl.ds(off, CH)], rowb.at[b], sems[b]),
            pltpu.make_async_copy(col_hbm.at[pl.ds(off, CH)], colb.at[b], sems[b]),
            pltpu.make_async_copy(val_hbm.at[pl.ds(off, CH)], valb.at[b], sems[b]),
        )

    for cp in _copies(0, 0):
        cp.start()

    pltpu.sync_copy(tab_hbm.at[pl.ds(e0, FPW * N_NODES)], tab_v)

    zero = jnp.zeros((LANES,), jnp.float32)

    def _zbody(i, carry):
        for u in range(4):
            acc_v[pl.ds((i * 4 + u) * LANES, LANES)] = zero
        return carry

    lax.fori_loop(0, FPW * N_NODES // (4 * LANES), _zbody, 0)

    def _process(b):
        def _gbody(q, carry):
            base = q * (LANES * UNROLL)
            for u in range(UNROLL):
                sl = pl.ds(base + u * LANES, LANES)
                row = rowb[b, sl]
                col = colb[b, sl]
                val = valb[b, sl]
                for d in range(FPW):
                    gidx = col if d == 0 else col + (d * N_NODES)
                    sidx = row if d == 0 else row + (d * N_NODES)
                    gathered = plsc.load_gather(tab_v, [gidx])
                    plsc.addupdate_scatter(acc_v, [sidx], gathered * val)
            return carry

        lax.fori_loop(0, GR // UNROLL, _gbody, 0)

    def _pair(i, carry):
        for b in range(2):
            ci = i * 2 + b

            @pl.when(ci + 1 < NCH)
            def _():
                for cp in _copies(ci + 1, 1 - b):
                    cp.start()

            for cp in _copies(ci, b):
                cp.wait()
            _process(b)
        return carry

    lax.fori_loop(0, NCH // 2, _pair, 0)

    pltpu.sync_copy(acc_v, out_hbm.at[pl.ds(e0, FPW * N_NODES)])


def _mm1_body(x_ref, w_ref, o_ref):
    o_ref[...] = lax.dot_general(
        w_ref[...], x_ref[...], (((0,), (1,)), ((), ())),
        preferred_element_type=jnp.float32)


def _dense1(x, W):
    return pl.pallas_call(
        _mm1_body,
        out_shape=jax.ShapeDtypeStruct((D_HID, N_NODES), jnp.float32),
    )(x, W)


def _mm2_body(p_ref, b_ref, w_ref, o_ref):
    h = jnp.maximum(p_ref[...] + b_ref[...], 0.0)
    o_ref[...] = lax.dot_general(
        w_ref[...], h, (((0,), (0,)), ((), ())),
        preferred_element_type=jnp.float32)


def _dense2(p, b, W):
    return pl.pallas_call(
        _mm2_body,
        out_shape=jax.ShapeDtypeStruct((D_HID, N_NODES), jnp.float32),
    )(p, b, W)


def _mm3_body(p_ref, b_ref, w_ref, cb_ref, o_ref):
    h2 = p_ref[...] + b_ref[...]
    logits = lax.dot_general(
        h2, w_ref[...], (((0,), (0,)), ((), ())),
        preferred_element_type=jnp.float32) + cb_ref[...]
    m = jnp.max(logits, axis=-1, keepdims=True)
    lse = jnp.log(jnp.sum(jnp.exp(logits - m), axis=-1, keepdims=True)) + m
    o_ref[...] = logits - lse


def _dense3(p, b, W, cb):
    return pl.pallas_call(
        _mm3_body,
        out_shape=jax.ShapeDtypeStruct((N_NODES, N_CLASSES), jnp.float32),
    )(p, b, W, cb)


def kernel(x, edge_index, adj_values, gc1_W, gc1_b, gc2_W, gc2_b, cls_W, cls_b):
    row = edge_index[0].astype(jnp.int32)
    col = edge_index[1].astype(jnp.int32)
    val = adj_values.astype(jnp.float32)
    b1 = gc1_b.reshape(D_HID, 1)
    b2 = gc2_b.reshape(D_HID, 1)
    cb = cls_b.reshape(1, N_CLASSES)

    s1T = _dense1(x, gc1_W)
    p1 = _spmm_sc(s1T.reshape(-1), row, col, val).reshape(D_HID, N_NODES)
    s2T = _dense2(p1, b1, gc2_W)
    p2 = _spmm_sc(s2T.reshape(-1), row, col, val).reshape(D_HID, N_NODES)
    return _dense3(p2, b2, cls_W, cb)


# speedup vs baseline: 6.4725x; 2.3945x over previous
import functools

import jax
import jax.numpy as jnp
from jax import lax
from jax.experimental import pallas as pl
from jax.experimental.pallas import tpu as pltpu
from jax.experimental.pallas import tpu_sc as plsc

N_NODES = 10000
N_EDGES = 320000
D_IN = 128
D_HID = 128
N_CLASSES = 40

NC = 2
NS = 16
LANES = 16
NW = NC * NS
FPW = D_HID // NW
CH = 3200
NCH = N_EDGES // CH
GR = CH // LANES

_SC_MESH = plsc.VectorSubcoreMesh(
    core_axis_name="c", subcore_axis_name="s", num_cores=NC, num_subcores=NS)


UNROLL = 4


@functools.partial(
    pl.kernel,
    out_type=jax.ShapeDtypeStruct((D_HID * N_NODES,), jnp.float32),
    mesh=_SC_MESH,
    compiler_params=pltpu.CompilerParams(needs_layout_passes=False),
    scratch_types=[
        pltpu.VMEM((FPW * N_NODES,), jnp.float32),
        pltpu.VMEM((FPW * N_NODES,), jnp.float32),
        pltpu.VMEM((2, CH), jnp.int32),
        pltpu.VMEM((2, CH), jnp.int32),
        pltpu.VMEM((2, CH), jnp.float32),
        pltpu.SemaphoreType.DMA,
        pltpu.SemaphoreType.DMA,
    ],
)
def _spmm_sc(tab_hbm, row_hbm, col_hbm, val_hbm, out_hbm,
             tab_v, acc_v, rowb, colb, valb, sem0, sem1):
    c = lax.axis_index("c")
    s = lax.axis_index("s")
    wid = s * NC + c
    e0 = wid * (FPW * N_NODES)
    sems = (sem0, sem1)

    def _copies(ci, b):
        off = ci * CH
        return (
            pltpu.make_async_copy(row_hbm.at[pl.ds(off, CH)], rowb.at[b], sems[b]),
            pltpu.make_async_copy(col_hbm.at[pl.ds(off, CH)], colb.at[b], sems[b]),
            pltpu.make_async_copy(val_hbm.at[pl.ds(off, CH)], valb.at[b], sems[b]),
        )

    for cp in _copies(0, 0):
        cp.start()

    pltpu.sync_copy(tab_hbm.at[pl.ds(e0, FPW * N_NODES)], tab_v)

    zero = jnp.zeros((LANES,), jnp.float32)

    @plsc.parallel_loop(0, FPW * N_NODES // LANES, unroll=8)
    def _zbody(i):
        acc_v[pl.ds(i * LANES, LANES)] = zero

    def _process(b):
        @plsc.parallel_loop(0, GR, unroll=UNROLL)
        def _gbody(g):
            sl = pl.ds(g * LANES, LANES)
            row = rowb[b, sl]
            col = colb[b, sl]
            val = valb[b, sl]
            for d in range(FPW):
                gidx = col if d == 0 else col + (d * N_NODES)
                sidx = row if d == 0 else row + (d * N_NODES)
                gathered = plsc.load_gather(tab_v, [gidx])
                plsc.addupdate_scatter(acc_v, [sidx], gathered * val)

    def _pair(i, carry):
        for b in range(2):
            ci = i * 2 + b

            @pl.when(ci + 1 < NCH)
            def _():
                for cp in _copies(ci + 1, 1 - b):
                    cp.start()

            for cp in _copies(ci, b):
                cp.wait()
            _process(b)
        return carry

    lax.fori_loop(0, NCH // 2, _pair, 0)

    pltpu.sync_copy(acc_v, out_hbm.at[pl.ds(e0, FPW * N_NODES)])


def _mm1_body(x_ref, w_ref, o_ref):
    o_ref[...] = lax.dot_general(
        w_ref[...], x_ref[...], (((0,), (1,)), ((), ())),
        preferred_element_type=jnp.float32)


def _dense1(x, W):
    return pl.pallas_call(
        _mm1_body,
        out_shape=jax.ShapeDtypeStruct((D_HID, N_NODES), jnp.float32),
    )(x, W)


def _mm2_body(p_ref, b_ref, w_ref, o_ref):
    h = jnp.maximum(p_ref[...] + b_ref[...], 0.0)
    o_ref[...] = lax.dot_general(
        w_ref[...], h, (((0,), (0,)), ((), ())),
        preferred_element_type=jnp.float32)


def _dense2(p, b, W):
    return pl.pallas_call(
        _mm2_body,
        out_shape=jax.ShapeDtypeStruct((D_HID, N_NODES), jnp.float32),
    )(p, b, W)


def _mm3_body(p_ref, b_ref, w_ref, cb_ref, o_ref):
    h2 = p_ref[...] + b_ref[...]
    logits = lax.dot_general(
        h2, w_ref[...], (((0,), (0,)), ((), ())),
        preferred_element_type=jnp.float32) + cb_ref[...]
    m = jnp.max(logits, axis=-1, keepdims=True)
    lse = jnp.log(jnp.sum(jnp.exp(logits - m), axis=-1, keepdims=True)) + m
    o_ref[...] = logits - lse


def _dense3(p, b, W, cb):
    return pl.pallas_call(
        _mm3_body,
        out_shape=jax.ShapeDtypeStruct((N_NODES, N_CLASSES), jnp.float32),
    )(p, b, W, cb)


def kernel(x, edge_index, adj_values, gc1_W, gc1_b, gc2_W, gc2_b, cls_W, cls_b):
    row = edge_index[0].astype(jnp.int32)
    col = edge_index[1].astype(jnp.int32)
    val = adj_values.astype(jnp.float32)
    b1 = gc1_b.reshape(D_HID, 1)
    b2 = gc2_b.reshape(D_HID, 1)
    cb = cls_b.reshape(1, N_CLASSES)

    s1T = _dense1(x, gc1_W)
    p1 = _spmm_sc(s1T.reshape(-1), row, col, val).reshape(D_HID, N_NODES)
    s2T = _dense2(p1, b1, gc2_W)
    p2 = _spmm_sc(s2T.reshape(-1), row, col, val).reshape(D_HID, N_NODES)
    return _dense3(p2, b2, cls_W, cb)
